# speedup vs baseline: 6.7264x; 6.7264x over previous
import functools

import jax
import jax.numpy as jnp
from jax import lax
from jax.experimental import pallas as pl
from jax.experimental.pallas import tpu as pltpu
from jax.experimental.pallas import tpu_sc as plsc

_N = 10000
_K = 16
_CIN = 128
_COUT = 128
_M = 9
_MP = 16
_NW = 32
_CHUNK = 128
_EDGES_PAD = 163840
_BN = 400
_GRID = _N // _BN


def _sc_gather(table, idx):
    n_idx = idx.shape[0]
    per_w = n_idx // _NW

    mesh = plsc.VectorSubcoreMesh(core_axis_name="c", subcore_axis_name="s")

    @functools.partial(
        pl.kernel,
        out_type=jax.ShapeDtypeStruct((n_idx, _CIN), jnp.float32),
        mesh=mesh,
        scratch_types=[
            pltpu.VMEM((_CHUNK,), jnp.int32),
            pltpu.VMEM((_CHUNK, _CIN), jnp.float32),
            pltpu.SemaphoreType.DMA,
        ],
    )
    def gather_kernel(table_hbm, idx_hbm, out_hbm, idx_v, rows_v, sem):
        wid = lax.axis_index("s") * 2 + lax.axis_index("c")
        base = wid * per_w

        @pl.loop(0, per_w // _CHUNK)
        def _(i):
            off = base + i * _CHUNK
            pltpu.sync_copy(idx_hbm.at[pl.ds(off, _CHUNK)], idx_v)
            pltpu.async_copy(table_hbm.at[idx_v], rows_v, sem).wait()
            pltpu.sync_copy(rows_v, out_hbm.at[pl.ds(off, _CHUNK)])

    return gather_kernel(table, idx)


def _tc_body(x_ref, adj_ref, xg_ref, ut_ref, vt_ref, c_ref, wt_ref, b_ref,
             o_ref):
    xb = x_ref[...]
    xg = xg_ref[...]
    adjb = adj_ref[...]

    ux = jnp.dot(xb, ut_ref[...], preferred_element_type=jnp.float32)
    xg2 = xg.reshape(_BN * _K, _CIN)
    vpat = jnp.dot(xg2, vt_ref[...], preferred_element_type=jnp.float32)
    logits = (ux[:, None, :] + vpat.reshape(_BN, _K, _MP)
              + c_ref[0][None, None, :])
    mx = jnp.max(logits, axis=-1, keepdims=True)
    e = jnp.exp(logits - mx)
    q = e / jnp.sum(e, axis=-1, keepdims=True)

    acc = jnp.zeros((_BN, _COUT), jnp.float32)
    for m in range(_M):
        ym = jnp.sum(xg * q[:, :, m][:, :, None], axis=1)
        acc = acc + jnp.dot(ym, wt_ref[m], preferred_element_type=jnp.float32)

    cnt = jnp.sum((adjb != 0).astype(jnp.float32), axis=1, keepdims=True)
    recip = jnp.where(cnt != 0.0, 1.0 / jnp.maximum(cnt, 1.0), 0.0)
    o_ref[...] = acc * recip + b_ref[0][None, :]


def kernel(x, adj, W, b, u, v, c):
    x2 = x[0]
    xpad = jnp.concatenate(
        [jnp.zeros((1, _CIN), jnp.float32), x2], axis=0)
    adj_flat = adj.reshape(-1).astype(jnp.int32)
    adj_p = jnp.concatenate(
        [adj_flat, jnp.zeros((_EDGES_PAD - _N * _K,), jnp.int32)])

    xg = _sc_gather(xpad, adj_p)
    xg3 = xg.reshape(_EDGES_PAD // _K, _K, _CIN)

    ut = jnp.pad(u.T, ((0, 0), (0, _MP - _M)))
    vt = jnp.pad(v.T, ((0, 0), (0, _MP - _M)))
    cb = jnp.broadcast_to(
        jnp.pad(c, (0, _MP - _M), constant_values=-1e30)[None, :], (8, _MP))
    wt = W.transpose(0, 2, 1)
    bb = jnp.broadcast_to(b[None, :], (8, _COUT))

    out = pl.pallas_call(
        _tc_body,
        grid=(_GRID,),
        in_specs=[
            pl.BlockSpec((_BN, _CIN), lambda i: (i, 0)),
            pl.BlockSpec((_BN, _K), lambda i: (i, 0)),
            pl.BlockSpec((_BN, _K, _CIN), lambda i: (i, 0, 0)),
            pl.BlockSpec((_CIN, _MP), lambda i: (0, 0)),
            pl.BlockSpec((_CIN, _MP), lambda i: (0, 0)),
            pl.BlockSpec((8, _MP), lambda i: (0, 0)),
            pl.BlockSpec((_M, _CIN, _COUT), lambda i: (0, 0, 0)),
            pl.BlockSpec((8, _COUT), lambda i: (0, 0)),
        ],
        out_specs=pl.BlockSpec((_BN, _COUT), lambda i: (i, 0)),
        out_shape=jax.ShapeDtypeStruct((_N, _COUT), jnp.float32),
    )(x2, adj[0], xg3, ut, vt, cb, wt, bb)

    return out[None]


# speedup vs baseline: 7.7079x; 1.1459x over previous
import functools

import jax
import jax.numpy as jnp
from jax import lax
from jax.experimental import pallas as pl
from jax.experimental.pallas import tpu as pltpu
from jax.experimental.pallas import tpu_sc as plsc

_N = 10000
_K = 16
_CIN = 128
_COUT = 128
_M = 9
_MP = 16
_NW = 32
_CHUNK = 128
_EDGES_PAD = 163840
_STEPS = _EDGES_PAD // (_NW * _CHUNK)
_BN = 400
_GRID = _N // _BN


_NB = 4


def _sc_gather(table, idx2):
    steps = idx2.shape[0] // _NW
    per_w = steps * _CHUNK
    n_idx = idx2.shape[0] * _CHUNK

    mesh = plsc.VectorSubcoreMesh(core_axis_name="c", subcore_axis_name="s")

    @functools.partial(
        pl.kernel,
        out_type=jax.ShapeDtypeStruct((n_idx, _CIN), jnp.float32),
        mesh=mesh,
        scratch_types=[
            pltpu.VMEM((steps, _CHUNK), jnp.int32),
            pltpu.VMEM((_NB, _CHUNK, _CIN), jnp.float32),
            pltpu.SemaphoreType.DMA((_NB,)),
            pltpu.SemaphoreType.DMA((_NB,)),
        ],
    )
    def gather_kernel(table_hbm, idx_hbm, out_hbm, idx_all, rows, gsem, ssem):
        wid = lax.axis_index("s") * 2 + lax.axis_index("c")
        base = wid * per_w

        pltpu.sync_copy(idx_hbm.at[pl.ds(wid * steps, steps)], idx_all)

        def start_gather(step, b):
            pltpu.async_copy(table_hbm.at[idx_all.at[step]], rows.at[b],
                             gsem.at[b])

        def wait_gather(b):
            pltpu.make_async_copy(table_hbm.at[idx_all.at[0]], rows.at[b],
                                  gsem.at[b]).wait()

        def start_store(step, b):
            off = base + step * _CHUNK
            pltpu.async_copy(rows.at[b], out_hbm.at[pl.ds(off, _CHUNK)],
                             ssem.at[b])

        def wait_store(b):
            pltpu.make_async_copy(rows.at[b],
                                  out_hbm.at[pl.ds(base, _CHUNK)],
                                  ssem.at[b]).wait()

        for b in range(_NB):
            start_gather(b, b)

        @pl.loop(0, steps // _NB - 1)
        def _(g):
            for b in range(_NB):
                i = g * _NB + b
                wait_gather(b)
                start_store(i, b)
                wait_store(b)
                start_gather(i + _NB, b)

        for b in range(_NB):
            wait_gather(b)
            start_store(steps - _NB + b, b)
            wait_store(b)

    return gather_kernel(table, idx2)


def _tc_body(x_ref, adj_ref, xg_ref, ut_ref, vt_ref, c_ref, wt_ref, b_ref,
             o_ref):
    xb = x_ref[...]
    xg = xg_ref[...]
    adjb = adj_ref[...]

    ux = jnp.dot(xb, ut_ref[...], preferred_element_type=jnp.float32)
    xg2 = xg.reshape(_BN * _K, _CIN)
    vpat = jnp.dot(xg2, vt_ref[...], preferred_element_type=jnp.float32)
    logits = (ux[:, None, :] + vpat.reshape(_BN, _K, _MP)
              + c_ref[0][None, None, :])
    mx = jnp.max(logits, axis=-1, keepdims=True)
    e = jnp.exp(logits - mx)
    q = e / jnp.sum(e, axis=-1, keepdims=True)

    acc = jnp.zeros((_BN, _COUT), jnp.float32)
    for m in range(_M):
        ym = jnp.sum(xg * q[:, :, m][:, :, None], axis=1)
        acc = acc + jnp.dot(ym, wt_ref[m], preferred_element_type=jnp.float32)

    cnt = jnp.sum((adjb != 0).astype(jnp.float32), axis=1, keepdims=True)
    recip = jnp.where(cnt != 0.0, 1.0 / jnp.maximum(cnt, 1.0), 0.0)
    o_ref[...] = acc * recip + b_ref[0][None, :]


def kernel(x, adj, W, b, u, v, c):
    x2 = x[0]
    xpad = jnp.concatenate(
        [jnp.zeros((1, _CIN), jnp.float32), x2], axis=0)
    adj_flat = adj.reshape(-1).astype(jnp.int32)
    adj_p = jnp.concatenate(
        [adj_flat, jnp.zeros((_EDGES_PAD - _N * _K,), jnp.int32)])

    xg = _sc_gather(xpad, adj_p.reshape(_NW * _STEPS, _CHUNK))
    xg3 = xg.reshape(_EDGES_PAD // _K, _K, _CIN)

    ut = jnp.pad(u.T, ((0, 0), (0, _MP - _M)))
    vt = jnp.pad(v.T, ((0, 0), (0, _MP - _M)))
    cb = jnp.broadcast_to(
        jnp.pad(c, (0, _MP - _M), constant_values=-1e30)[None, :], (8, _MP))
    wt = W.transpose(0, 2, 1)
    bb = jnp.broadcast_to(b[None, :], (8, _COUT))

    out = pl.pallas_call(
        _tc_body,
        grid=(_GRID,),
        in_specs=[
            pl.BlockSpec((_BN, _CIN), lambda i: (i, 0)),
            pl.BlockSpec((_BN, _K), lambda i: (i, 0)),
            pl.BlockSpec((_BN, _K, _CIN), lambda i: (i, 0, 0)),
            pl.BlockSpec((_CIN, _MP), lambda i: (0, 0)),
            pl.BlockSpec((_CIN, _MP), lambda i: (0, 0)),
            pl.BlockSpec((8, _MP), lambda i: (0, 0)),
            pl.BlockSpec((_M, _CIN, _COUT), lambda i: (0, 0, 0)),
            pl.BlockSpec((8, _COUT), lambda i: (0, 0)),
        ],
        out_specs=pl.BlockSpec((_BN, _COUT), lambda i: (i, 0)),
        out_shape=jax.ShapeDtypeStruct((_N, _COUT), jnp.float32),
    )(x2, adj[0], xg3, ut, vt, cb, wt, bb)

    return out[None]


# speedup vs baseline: 10.3885x; 1.3478x over previous
import functools

import jax
import jax.numpy as jnp
from jax import lax
from jax.experimental import pallas as pl
from jax.experimental.pallas import tpu as pltpu
from jax.experimental.pallas import tpu_sc as plsc

_N = 10000
_K = 16
_CIN = 128
_COUT = 128
_M = 9
_MP = 16
_NW = 32
_CHUNK = 128
_NPAD = 10240
_BN = 400
_GRID = _N // _BN
_NB = 5


def _sc_gather(table, idx2):
    n_chunks = idx2.shape[0]
    steps = n_chunks // _NW
    idx3 = idx2.reshape(_NW, steps, _CHUNK)
    dt = table.dtype
    width = table.shape[1]
    mesh = plsc.VectorSubcoreMesh(core_axis_name="c", subcore_axis_name="s")

    @functools.partial(
        pl.kernel,
        out_type=jax.ShapeDtypeStruct((n_chunks * _CHUNK, width), dt),
        mesh=mesh,
        scratch_types=[
            pltpu.VMEM((steps, _CHUNK), jnp.int32),
            pltpu.VMEM((_NB, _CHUNK, width), dt),
            pltpu.SemaphoreType.DMA((_NB,)),
            pltpu.SemaphoreType.DMA((_NB,)),
        ],
    )
    def gather_kernel(table_hbm, idx_hbm, out_hbm, idx_all, rows, gsem, ssem):
        wid = lax.axis_index("s") * 2 + lax.axis_index("c")
        j0 = wid * steps

        pltpu.sync_copy(idx_hbm.at[wid], idx_all)

        def start_gather(step, b):
            pltpu.async_copy(table_hbm.at[idx_all.at[step]], rows.at[b],
                             gsem.at[b])

        def wait_gather(b):
            pltpu.make_async_copy(table_hbm.at[idx_all.at[0]], rows.at[b],
                                  gsem.at[b]).wait()

        def start_store(step, b):
            off = (j0 + step) * _CHUNK
            pltpu.async_copy(rows.at[b], out_hbm.at[pl.ds(off, _CHUNK)],
                             ssem.at[b])

        def wait_store(b):
            pltpu.make_async_copy(rows.at[b],
                                  out_hbm.at[pl.ds(0, _CHUNK)],
                                  ssem.at[b]).wait()

        for b in range(_NB):
            start_gather(b, b)

        @pl.loop(0, steps // _NB - 1)
        def _(g):
            for b in range(_NB):
                i = g * _NB + b
                wait_gather(b)
                start_store(i, b)
                wait_store(b)
                start_gather(i + _NB, b)

        for b in range(_NB):
            wait_gather(b)
            start_store(steps - _NB + b, b)
            wait_store(b)

    return gather_kernel(table, idx3)


def _partial_sum(x_ref, xg_ref, ut_ref, vt_ref, c_ref, wt_ref, o_ref):
    kk = xg_ref.shape[0]
    xb = x_ref[...]
    xgt = xg_ref[...].astype(jnp.bfloat16)

    ux = jnp.dot(xb, ut_ref[...], preferred_element_type=jnp.float32)
    xg2 = xgt.reshape(kk * _BN, _CIN)
    vpat = jnp.dot(xg2, vt_ref[...], preferred_element_type=jnp.float32)
    logits = (ux[None, :, :] + vpat.reshape(kk, _BN, _MP)
              + c_ref[0][None, None, :])
    mx = jnp.max(logits, axis=-1, keepdims=True)
    e = jnp.exp(logits - mx)
    q = (e / jnp.sum(e, axis=-1, keepdims=True)).astype(jnp.bfloat16)

    acc = jnp.zeros((_BN, _COUT), jnp.float32)
    for m in range(_M):
        t = xgt * q[:, :, m][:, :, None]
        ym = jnp.sum(t, axis=0)
        acc = acc + jnp.dot(ym, wt_ref[m], preferred_element_type=jnp.float32)
    o_ref[...] = acc


def _tc_a(x_ref, xg_ref, ut_ref, vt_ref, c_ref, wt_ref, o_ref):
    _partial_sum(x_ref, xg_ref, ut_ref, vt_ref, c_ref, wt_ref, o_ref)


def _tc_b(x_ref, xg_ref, ut_ref, vt_ref, c_ref, wt_ref, adj_ref, b_ref,
          p0_ref, p1_ref, p2_ref, o_ref):
    _partial_sum(x_ref, xg_ref, ut_ref, vt_ref, c_ref, wt_ref, o_ref)
    adjb = adj_ref[...]
    cnt = jnp.sum((adjb != 0).astype(jnp.float32), axis=1, keepdims=True)
    recip = jnp.where(cnt != 0.0, 1.0 / jnp.maximum(cnt, 1.0), 0.0)
    psum = o_ref[...] + p0_ref[...] + p1_ref[...] + p2_ref[...]
    o_ref[...] = psum * recip + b_ref[0][None, :]


_COMMON_SPECS = [
    pl.BlockSpec((_BN, _CIN), lambda i: (i, 0)),
    pl.BlockSpec((_K // 4, _BN, _CIN), lambda i: (0, i, 0)),
    pl.BlockSpec((_CIN, _MP), lambda i: (0, 0)),
    pl.BlockSpec((_CIN, _MP), lambda i: (0, 0)),
    pl.BlockSpec((8, _MP), lambda i: (0, 0)),
    pl.BlockSpec((_M, _CIN, _COUT), lambda i: (0, 0, 0)),
]


def kernel(x, adj, W, b, u, v, c):
    x2 = x[0]
    xpad = jnp.concatenate(
        [jnp.zeros((1, _CIN), jnp.float32), x2], axis=0)
    adjt_p = jnp.pad(adj[0].T.astype(jnp.int32), ((0, 0), (0, _NPAD - _N)))
    idx2 = adjt_p.reshape(-1, _CHUNK)
    half_chunks = idx2.shape[0] // 2

    ut = jnp.pad(u.T, ((0, 0), (0, _MP - _M)))
    vt = jnp.pad(v.T, ((0, 0), (0, _MP - _M))).astype(jnp.bfloat16)
    cb = jnp.broadcast_to(
        jnp.pad(c, (0, _MP - _M), constant_values=-1e30)[None, :], (8, _MP))
    wt = W.transpose(0, 2, 1).astype(jnp.bfloat16)
    bb = jnp.broadcast_to(b[None, :], (8, _COUT))

    quarter = idx2.shape[0] // 4
    xgs = [_sc_gather(xpad, idx2[i * quarter:(i + 1) * quarter])
           .reshape(_K // 4, _NPAD, _CIN) for i in range(4)]

    parts = [pl.pallas_call(
        _tc_a,
        grid=(_GRID,),
        in_specs=_COMMON_SPECS,
        out_specs=pl.BlockSpec((_BN, _COUT), lambda i: (i, 0)),
        out_shape=jax.ShapeDtypeStruct((_N, _COUT), jnp.float32),
    )(x2, xgs[p], ut, vt, cb, wt) for p in range(3)]

    out = pl.pallas_call(
        _tc_b,
        grid=(_GRID,),
        in_specs=_COMMON_SPECS + [
            pl.BlockSpec((_BN, _K), lambda i: (i, 0)),
            pl.BlockSpec((8, _COUT), lambda i: (0, 0)),
            pl.BlockSpec((_BN, _COUT), lambda i: (i, 0)),
            pl.BlockSpec((_BN, _COUT), lambda i: (i, 0)),
            pl.BlockSpec((_BN, _COUT), lambda i: (i, 0)),
        ],
        out_specs=pl.BlockSpec((_BN, _COUT), lambda i: (i, 0)),
        out_shape=jax.ShapeDtypeStruct((_N, _COUT), jnp.float32),
    )(x2, xgs[3], ut, vt, cb, wt, adj[0], bb, *parts)

    return out[None]
